# baseline (speedup 1.0000x reference)
import functools

import jax
import jax.numpy as jnp
from jax import lax
from jax.experimental import pallas as pl
from jax.experimental.pallas import tpu as pltpu
from jax.experimental.pallas import tpu_sc as plsc

N = 10000
E = 320000
D = 128

NC = 2
NS = 16
NW = NC * NS
C = 128
NCHUNK = 80
EPW = NCHUNK * C
E_PAD = NW * EPW
NPAIR = NCHUNK // 2
NP = 10240
ZR = NP // NS

_sc_mesh = plsc.VectorSubcoreMesh(
    core_axis_name="c", subcore_axis_name="s", num_cores=NC, num_subcores=NS)


@functools.partial(
    pl.kernel,
    out_type=jax.ShapeDtypeStruct((NC, NP, D), jnp.float32),
    mesh=_sc_mesh,
    scratch_types=[
        pltpu.VMEM((C,), jnp.int32),
        pltpu.VMEM((C,), jnp.int32),
        pltpu.VMEM((NCHUNK, C), jnp.int32),
        pltpu.VMEM((C, D), jnp.float32),
        pltpu.VMEM((C, D), jnp.float32),
        pltpu.VMEM_SHARED((NP, D), jnp.float32),
        pltpu.SemaphoreType.DMA,
        pltpu.SemaphoreType.DMA,
        pltpu.SemaphoreType.DMA,
        pltpu.SemaphoreType.DMA,
    ],
)
def _sc_agg(h_hbm, src_hbm, dst_hbm, zero_hbm, out_hbm,
            sidx0, sidx1, didx, rows0, rows1, acc,
            isem0, isem1, gsem0, gsem1):
    cid = lax.axis_index("c")
    sid = lax.axis_index("s")
    wid = cid * NS + sid
    pltpu.sync_copy(dst_hbm.at[wid], didx)
    pltpu.sync_copy(zero_hbm.at[pl.ds(sid * ZR, ZR)], acc.at[pl.ds(sid * ZR, ZR)])
    plsc.subcore_barrier()

    def iload(j, buf, sem):
        pltpu.async_copy(src_hbm.at[pl.ds(wid * EPW + j * C, C)], buf, sem)

    def iwait(j, buf, sem):
        pltpu.make_async_copy(
            src_hbm.at[pl.ds(wid * EPW + j * C, C)], buf, sem).wait()

    def gwait(buf, sem):
        pltpu.make_async_copy(h_hbm.at[sidx0], buf, sem).wait()

    iload(0, sidx0, isem0)
    iwait(0, sidx0, isem0)
    pltpu.async_copy(h_hbm.at[sidx0], rows0, gsem0)
    iload(1, sidx1, isem1)

    def step(k, carry):
        j0 = 2 * k
        iwait(j0 + 1, sidx1, isem1)
        pltpu.async_copy(h_hbm.at[sidx1], rows1, gsem1)
        gwait(rows0, gsem0)
        pltpu.sync_copy(rows0, acc.at[didx.at[j0]], add=True)
        iload(j0 + 2, sidx0, isem0)
        iwait(j0 + 2, sidx0, isem0)
        pltpu.async_copy(h_hbm.at[sidx0], rows0, gsem0)
        gwait(rows1, gsem1)
        pltpu.sync_copy(rows1, acc.at[didx.at[j0 + 1]], add=True)
        iload(j0 + 3, sidx1, isem1)
        return carry

    lax.fori_loop(0, NPAIR - 1, step, 0)
    j0 = NCHUNK - 2
    iwait(j0 + 1, sidx1, isem1)
    pltpu.async_copy(h_hbm.at[sidx1], rows1, gsem1)
    gwait(rows0, gsem0)
    pltpu.sync_copy(rows0, acc.at[didx.at[j0]], add=True)
    gwait(rows1, gsem1)
    pltpu.sync_copy(rows1, acc.at[didx.at[j0 + 1]], add=True)

    plsc.subcore_barrier()
    pltpu.sync_copy(acc.at[pl.ds(sid * ZR, ZR)],
                    out_hbm.at[cid, pl.ds(sid * ZR, ZR)])


@functools.partial(
    pl.kernel,
    out_type=jax.ShapeDtypeStruct((NC, NP, D), jnp.float32),
    mesh=_sc_mesh,
    scratch_types=[
        pltpu.VMEM((NCHUNK, C), jnp.int32),
        pltpu.VMEM((C, D), jnp.float32),
        pltpu.VMEM_SHARED((NP, D), jnp.float32),
    ],
)
def _sc_count(dst_hbm, zero_hbm, out_hbm, didx, ones_v, acc):
    cid = lax.axis_index("c")
    sid = lax.axis_index("s")
    wid = cid * NS + sid
    pltpu.sync_copy(dst_hbm.at[wid], didx)
    pltpu.sync_copy(zero_hbm.at[pl.ds(sid * ZR, ZR)], acc.at[pl.ds(sid * ZR, ZR)])

    one = jnp.ones((16,), jnp.float32)

    def fill(i, carry):
        for c16 in range(D // 16):
            ones_v[i, pl.ds(c16 * 16, 16)] = one
        return carry

    lax.fori_loop(0, C, fill, 0)
    plsc.subcore_barrier()

    def step(j, carry):
        pltpu.sync_copy(ones_v, acc.at[didx.at[j]], add=True)
        return carry

    lax.fori_loop(0, NCHUNK, step, 0)
    plsc.subcore_barrier()
    pltpu.sync_copy(acc.at[pl.ds(sid * ZR, ZR)],
                    out_hbm.at[cid, pl.ds(sid * ZR, ZR)])


_RB = 1000


def _combine_body(has_bn, h, accp, cntp, w1, w2, b, g, be, out):
    s = accp[0] + accp[1]
    c = cntp[0, :, 0:1] + cntp[1, :, 0:1]
    hn = s * (1.0 / jnp.maximum(c, 1.0))
    h2 = (lax.dot_general(h[...], w1[...], (((1,), (1,)), ((), ())),
                          preferred_element_type=jnp.float32)
          + lax.dot_general(hn, w2[...], (((1,), (1,)), ((), ())),
                            preferred_element_type=jnp.float32)
          + b[...])
    nrm = jnp.maximum(jnp.sqrt(jnp.sum(h2 * h2, axis=1, keepdims=True)), 1e-12)
    y = h2 / nrm
    if has_bn:
        y = y * (g[...] / jnp.sqrt(1.0 + 1e-5)) + be[...]
        y = jnp.maximum(y, 0.0)
    out[...] = y


def _combine(h, acc, cnt, w1, w2, b, g, be, has_bn):
    mat = pl.BlockSpec((D, D), lambda i: (0, 0))
    vec = pl.BlockSpec((1, D), lambda i: (0, 0))
    return pl.pallas_call(
        functools.partial(_combine_body, has_bn),
        grid=(N // _RB,),
        in_specs=[
            pl.BlockSpec((_RB, D), lambda i: (i, 0)),
            pl.BlockSpec((NC, _RB, D), lambda i: (0, i, 0)),
            pl.BlockSpec((NC, _RB, 8), lambda i: (0, i, 0)),
            mat, mat, vec, vec, vec,
        ],
        out_specs=pl.BlockSpec((_RB, D), lambda i: (i, 0)),
        out_shape=jax.ShapeDtypeStruct((N, D), jnp.float32),
    )(h, acc, cnt, w1, w2, b[None, :], g[None, :], be[None, :])


def kernel(x, edge_index, W1_0, W2_0, b_0, W1_1, W2_1, b_1, W1_2, W2_2, b_2,
           g_0, be_0, g_1, be_1):
    ei = edge_index.astype(jnp.int32)
    pad = E_PAD - E
    src = jnp.concatenate([ei[0], jnp.zeros((pad,), jnp.int32)])
    dst = jnp.concatenate([ei[1], jnp.full((pad,), NP - 1, jnp.int32)])
    dst3 = dst.reshape(NW, NCHUNK, C)
    zero_nd = jnp.zeros((NP, D), jnp.float32)

    cnt = _sc_count(dst3, zero_nd)[:, :, :8]

    h = x
    layers = [
        (W1_0, W2_0, b_0, g_0, be_0, True),
        (W1_1, W2_1, b_1, g_1, be_1, True),
        (W1_2, W2_2, b_2, g_1, be_1, False),
    ]
    for w1, w2, b, g, be, has_bn in layers:
        acc = _sc_agg(h, src, dst3, zero_nd)
        h = _combine(h, acc, cnt, w1, w2, b, g, be, has_bn)
    return h


# speedup vs baseline: 1.5044x; 1.5044x over previous
import functools

import jax
import jax.numpy as jnp
from jax import lax
from jax.experimental import pallas as pl
from jax.experimental.pallas import tpu as pltpu
from jax.experimental.pallas import tpu_sc as plsc

N = 10000
E = 320000
D = 128

NC = 2
NS = 16
NW = NC * NS
C = 128
NCHUNK = 80
EPW = NCHUNK * C
E_PAD = NW * EPW
NPAIR = NCHUNK // 2
NP = 10240
ZR = NP // NS

_sc_mesh = plsc.VectorSubcoreMesh(
    core_axis_name="c", subcore_axis_name="s", num_cores=NC, num_subcores=NS)


AC = 80
ANCHUNK = 10000 // AC
AEPW = 10000


@functools.partial(
    pl.kernel,
    out_type=jax.ShapeDtypeStruct((NC, NP, D), jnp.float32),
    mesh=_sc_mesh,
    scratch_types=[
        pltpu.VMEM((AC,), jnp.int32),
        pltpu.VMEM((AC,), jnp.int32),
        pltpu.VMEM((AC, D), jnp.float32),
        pltpu.VMEM_SHARED((NP, D), jnp.float32),
        pltpu.SemaphoreType.DMA,
    ],
)
def _sc_agg(h_hbm, src_hbm, dst_hbm, zero_hbm, out_hbm, sidx, didx, rows, acc, sem):
    cid = lax.axis_index("c")
    sid = lax.axis_index("s")
    wid = cid * NS + sid
    pltpu.sync_copy(zero_hbm.at[pl.ds(sid * ZR, ZR)], acc.at[pl.ds(sid * ZR, ZR)])
    plsc.subcore_barrier()

    def step(i, carry):
        e0 = wid * AEPW + i * AC
        pltpu.sync_copy(src_hbm.at[pl.ds(e0, AC)], sidx)
        pltpu.sync_copy(dst_hbm.at[pl.ds(e0, AC)], didx)
        pltpu.async_copy(h_hbm.at[sidx], rows, sem).wait()
        pltpu.sync_copy(rows, acc.at[didx], add=True)
        return carry

    lax.fori_loop(0, ANCHUNK, step, 0)
    plsc.subcore_barrier()
    pltpu.sync_copy(acc.at[pl.ds(sid * ZR, ZR)],
                    out_hbm.at[cid, pl.ds(sid * ZR, ZR)])


@functools.partial(
    pl.kernel,
    out_type=jax.ShapeDtypeStruct((NC, NP, D), jnp.float32),
    mesh=_sc_mesh,
    scratch_types=[
        pltpu.VMEM((NCHUNK, C), jnp.int32),
        pltpu.VMEM((C, D), jnp.float32),
        pltpu.VMEM_SHARED((NP, D), jnp.float32),
    ],
)
def _sc_count(dst_hbm, zero_hbm, out_hbm, didx, ones_v, acc):
    cid = lax.axis_index("c")
    sid = lax.axis_index("s")
    wid = cid * NS + sid
    pltpu.sync_copy(dst_hbm.at[wid], didx)
    pltpu.sync_copy(zero_hbm.at[pl.ds(sid * ZR, ZR)], acc.at[pl.ds(sid * ZR, ZR)])

    one = jnp.ones((16,), jnp.float32)

    def fill(i, carry):
        for c16 in range(D // 16):
            ones_v[i, pl.ds(c16 * 16, 16)] = one
        return carry

    lax.fori_loop(0, C, fill, 0)
    plsc.subcore_barrier()

    def step(j, carry):
        pltpu.sync_copy(ones_v, acc.at[didx.at[j]], add=True)
        return carry

    lax.fori_loop(0, NCHUNK, step, 0)
    plsc.subcore_barrier()
    pltpu.sync_copy(acc.at[pl.ds(sid * ZR, ZR)],
                    out_hbm.at[cid, pl.ds(sid * ZR, ZR)])


_RB = 1000


def _combine_body(has_bn, h, accp, cntp, w1, w2, b, g, be, out):
    s = accp[0] + accp[1]
    c = cntp[0, :, 0:1] + cntp[1, :, 0:1]
    hn = s * (1.0 / jnp.maximum(c, 1.0))
    h2 = (lax.dot_general(h[...], w1[...], (((1,), (1,)), ((), ())),
                          preferred_element_type=jnp.float32)
          + lax.dot_general(hn, w2[...], (((1,), (1,)), ((), ())),
                            preferred_element_type=jnp.float32)
          + b[...])
    nrm = jnp.maximum(jnp.sqrt(jnp.sum(h2 * h2, axis=1, keepdims=True)), 1e-12)
    y = h2 / nrm
    if has_bn:
        y = y * (g[...] / jnp.sqrt(1.0 + 1e-5)) + be[...]
        y = jnp.maximum(y, 0.0)
    out[...] = y


def _combine(h, acc, cnt, w1, w2, b, g, be, has_bn):
    mat = pl.BlockSpec((D, D), lambda i: (0, 0))
    vec = pl.BlockSpec((1, D), lambda i: (0, 0))
    return pl.pallas_call(
        functools.partial(_combine_body, has_bn),
        grid=(N // _RB,),
        in_specs=[
            pl.BlockSpec((_RB, D), lambda i: (i, 0)),
            pl.BlockSpec((NC, _RB, D), lambda i: (0, i, 0)),
            pl.BlockSpec((NC, _RB, 8), lambda i: (0, i, 0)),
            mat, mat, vec, vec, vec,
        ],
        out_specs=pl.BlockSpec((_RB, D), lambda i: (i, 0)),
        out_shape=jax.ShapeDtypeStruct((N, D), jnp.float32),
    )(h, acc, cnt, w1, w2, b[None, :], g[None, :], be[None, :])


def kernel(x, edge_index, W1_0, W2_0, b_0, W1_1, W2_1, b_1, W1_2, W2_2, b_2,
           g_0, be_0, g_1, be_1):
    ei = edge_index.astype(jnp.int32)
    pad = E_PAD - E
    src = ei[0]
    dst = ei[1]
    dst3 = jnp.concatenate([dst, jnp.full((pad,), NP - 1, jnp.int32)]).reshape(
        NW, NCHUNK, C)
    zero_nd = jnp.zeros((NP, D), jnp.float32)

    cnt = _sc_count(dst3, zero_nd)[:, :, :8]

    h = x
    layers = [
        (W1_0, W2_0, b_0, g_0, be_0, True),
        (W1_1, W2_1, b_1, g_1, be_1, True),
        (W1_2, W2_2, b_2, g_1, be_1, False),
    ]
    for w1, w2, b, g, be, has_bn in layers:
        acc = _sc_agg(h, src, dst, zero_nd)
        h = _combine(h, acc, cnt, w1, w2, b, g, be, has_bn)
    return h


# speedup vs baseline: 2.3209x; 1.5427x over previous
import functools

import jax
import jax.numpy as jnp
from jax import lax
from jax.experimental import pallas as pl
from jax.experimental.pallas import tpu as pltpu
from jax.experimental.pallas import tpu_sc as plsc

N = 10000
E = 320000
D = 128

NC = 2
NS = 16
NW = NC * NS
C = 128
NCHUNK = 80
EPW = NCHUNK * C
E_PAD = NW * EPW
NPAIR = NCHUNK // 2
NP = 10240
ZR = NP // NS

_sc_mesh = plsc.VectorSubcoreMesh(
    core_axis_name="c", subcore_axis_name="s", num_cores=NC, num_subcores=NS)


AC = 80
ANCHUNK = 10000 // AC
AEPW = 10000


@functools.partial(
    pl.kernel,
    out_type=jax.ShapeDtypeStruct((NC, NP, D), jnp.float32),
    mesh=_sc_mesh,
    scratch_types=[
        pltpu.VMEM((AC,), jnp.int32),
        pltpu.VMEM((AC,), jnp.int32),
        pltpu.VMEM((AC,), jnp.int32),
        pltpu.VMEM((AC,), jnp.int32),
        pltpu.VMEM((AC, D), jnp.float32),
        pltpu.VMEM((AC, D), jnp.float32),
        pltpu.VMEM_SHARED((NP, D), jnp.float32),
        pltpu.SemaphoreType.DMA,
        pltpu.SemaphoreType.DMA,
    ],
)
def _sc_agg(h_hbm, src_hbm, dst_hbm, zero_hbm, out_hbm,
            sidx0, didx0, sidx1, didx1, rows0, rows1, acc, sem0, sem1):
    cid = lax.axis_index("c")
    sid = lax.axis_index("s")
    wid = cid * NS + sid
    pltpu.sync_copy(zero_hbm.at[pl.ds(sid * ZR, ZR)], acc.at[pl.ds(sid * ZR, ZR)])
    plsc.subcore_barrier()

    def load_and_gather(j, sidx, didx, rows, sem):
        e0 = wid * AEPW + j * AC
        pltpu.sync_copy(src_hbm.at[pl.ds(e0, AC)], sidx)
        pltpu.sync_copy(dst_hbm.at[pl.ds(e0, AC)], didx)
        pltpu.async_copy(h_hbm.at[sidx], rows, sem)

    def gwait(sidx, rows, sem):
        pltpu.make_async_copy(h_hbm.at[sidx], rows, sem).wait()

    load_and_gather(0, sidx0, didx0, rows0, sem0)

    def step(k, carry):
        load_and_gather(2 * k + 1, sidx1, didx1, rows1, sem1)
        gwait(sidx0, rows0, sem0)
        pltpu.sync_copy(rows0, acc.at[didx0], add=True)
        load_and_gather(2 * k + 2, sidx0, didx0, rows0, sem0)
        gwait(sidx1, rows1, sem1)
        pltpu.sync_copy(rows1, acc.at[didx1], add=True)
        return carry

    lax.fori_loop(0, (ANCHUNK - 1) // 2, step, 0)
    gwait(sidx0, rows0, sem0)
    pltpu.sync_copy(rows0, acc.at[didx0], add=True)
    plsc.subcore_barrier()
    pltpu.sync_copy(acc.at[pl.ds(sid * ZR, ZR)],
                    out_hbm.at[cid, pl.ds(sid * ZR, ZR)])


@functools.partial(
    pl.kernel,
    out_type=jax.ShapeDtypeStruct((NC, NP, D), jnp.float32),
    mesh=_sc_mesh,
    scratch_types=[
        pltpu.VMEM((NCHUNK, C), jnp.int32),
        pltpu.VMEM((C, D), jnp.float32),
        pltpu.VMEM_SHARED((NP, D), jnp.float32),
    ],
)
def _sc_count(dst_hbm, zero_hbm, out_hbm, didx, ones_v, acc):
    cid = lax.axis_index("c")
    sid = lax.axis_index("s")
    wid = cid * NS + sid
    pltpu.sync_copy(dst_hbm.at[wid], didx)
    pltpu.sync_copy(zero_hbm.at[pl.ds(sid * ZR, ZR)], acc.at[pl.ds(sid * ZR, ZR)])

    one = jnp.ones((16,), jnp.float32)

    def fill(i, carry):
        for c16 in range(D // 16):
            ones_v[i, pl.ds(c16 * 16, 16)] = one
        return carry

    lax.fori_loop(0, C, fill, 0)
    plsc.subcore_barrier()

    def step(j, carry):
        pltpu.sync_copy(ones_v, acc.at[didx.at[j]], add=True)
        return carry

    lax.fori_loop(0, NCHUNK, step, 0)
    plsc.subcore_barrier()
    pltpu.sync_copy(acc.at[pl.ds(sid * ZR, ZR)],
                    out_hbm.at[cid, pl.ds(sid * ZR, ZR)])


_RB = 1000


def _combine_body(has_bn, h, accp, cntp, w1, w2, b, g, be, out):
    s = accp[0] + accp[1]
    c = cntp[0, :, 0:1] + cntp[1, :, 0:1]
    hn = s * (1.0 / jnp.maximum(c, 1.0))
    h2 = (lax.dot_general(h[...], w1[...], (((1,), (1,)), ((), ())),
                          preferred_element_type=jnp.float32)
          + lax.dot_general(hn, w2[...], (((1,), (1,)), ((), ())),
                            preferred_element_type=jnp.float32)
          + b[...])
    nrm = jnp.maximum(jnp.sqrt(jnp.sum(h2 * h2, axis=1, keepdims=True)), 1e-12)
    y = h2 / nrm
    if has_bn:
        y = y * (g[...] / jnp.sqrt(1.0 + 1e-5)) + be[...]
        y = jnp.maximum(y, 0.0)
    out[...] = y


def _combine(h, acc, cnt, w1, w2, b, g, be, has_bn):
    mat = pl.BlockSpec((D, D), lambda i: (0, 0))
    vec = pl.BlockSpec((1, D), lambda i: (0, 0))
    return pl.pallas_call(
        functools.partial(_combine_body, has_bn),
        grid=(N // _RB,),
        in_specs=[
            pl.BlockSpec((_RB, D), lambda i: (i, 0)),
            pl.BlockSpec((NC, _RB, D), lambda i: (0, i, 0)),
            pl.BlockSpec((NC, _RB, 8), lambda i: (0, i, 0)),
            mat, mat, vec, vec, vec,
        ],
        out_specs=pl.BlockSpec((_RB, D), lambda i: (i, 0)),
        out_shape=jax.ShapeDtypeStruct((N, D), jnp.float32),
    )(h, acc, cnt, w1, w2, b[None, :], g[None, :], be[None, :])


def kernel(x, edge_index, W1_0, W2_0, b_0, W1_1, W2_1, b_1, W1_2, W2_2, b_2,
           g_0, be_0, g_1, be_1):
    ei = edge_index.astype(jnp.int32)
    pad = E_PAD - E
    src = ei[0]
    dst = ei[1]
    dst3 = jnp.concatenate([dst, jnp.full((pad,), NP - 1, jnp.int32)]).reshape(
        NW, NCHUNK, C)
    zero_nd = jnp.zeros((NP, D), jnp.float32)

    cnt = _sc_count(dst3, zero_nd)[:, :, :8]

    h = x
    layers = [
        (W1_0, W2_0, b_0, g_0, be_0, True),
        (W1_1, W2_1, b_1, g_1, be_1, True),
        (W1_2, W2_2, b_2, g_1, be_1, False),
    ]
    for w1, w2, b, g, be, has_bn in layers:
        acc = _sc_agg(h, src, dst, zero_nd)
        h = _combine(h, acc, cnt, w1, w2, b, g, be, has_bn)
    return h


# speedup vs baseline: 2.7407x; 1.1809x over previous
import functools

import jax
import jax.numpy as jnp
from jax import lax
from jax.experimental import pallas as pl
from jax.experimental.pallas import tpu as pltpu
from jax.experimental.pallas import tpu_sc as plsc

N = 10000
E = 320000
D = 128

NC = 2
NS = 16
NW = NC * NS
C = 128
NCHUNK = 80
EPW = NCHUNK * C
E_PAD = NW * EPW
NPAIR = NCHUNK // 2
NP = 10240
ZR = NP // NS

_sc_mesh = plsc.VectorSubcoreMesh(
    core_axis_name="c", subcore_axis_name="s", num_cores=NC, num_subcores=NS)


AC = 80
ANCHUNK = 10000 // AC
AEPW = 10000


@functools.partial(
    pl.kernel,
    out_type=jax.ShapeDtypeStruct((NC, NP, D), jnp.float32),
    mesh=_sc_mesh,
    scratch_types=[
        pltpu.VMEM((AC,), jnp.int32),
        pltpu.VMEM((AC,), jnp.int32),
        pltpu.VMEM((AC,), jnp.int32),
        pltpu.VMEM((AC,), jnp.int32),
        pltpu.VMEM((AC, D), jnp.float32),
        pltpu.VMEM((AC, D), jnp.float32),
        pltpu.VMEM_SHARED((NP, D), jnp.float32),
        pltpu.SemaphoreType.DMA,
        pltpu.SemaphoreType.DMA,
        pltpu.SemaphoreType.DMA,
        pltpu.SemaphoreType.DMA,
    ],
)
def _sc_agg(h_hbm, src_hbm, dst_hbm, zero_hbm, out_hbm,
            sidx0, didx0, sidx1, didx1, rows0, rows1, acc,
            sem0, sem1, isem0, isem1):
    cid = lax.axis_index("c")
    sid = lax.axis_index("s")
    wid = cid * NS + sid
    pltpu.sync_copy(zero_hbm.at[pl.ds(sid * ZR, ZR)], acc.at[pl.ds(sid * ZR, ZR)])
    plsc.subcore_barrier()

    def iload(j, sidx, didx, isem):
        e0 = wid * AEPW + j * AC
        pltpu.async_copy(src_hbm.at[pl.ds(e0, AC)], sidx, isem)
        pltpu.async_copy(dst_hbm.at[pl.ds(e0, AC)], didx, isem)

    def iwait(j, sidx, didx, isem):
        e0 = wid * AEPW + j * AC
        pltpu.make_async_copy(src_hbm.at[pl.ds(e0, AC)], sidx, isem).wait()
        pltpu.make_async_copy(dst_hbm.at[pl.ds(e0, AC)], didx, isem).wait()

    def gwait(sidx, rows, sem):
        pltpu.make_async_copy(h_hbm.at[sidx], rows, sem).wait()

    def scat(rows, didx):
        pltpu.sync_copy(rows, acc.at[didx], add=True)

    iload(0, sidx0, didx0, isem0)
    iwait(0, sidx0, didx0, isem0)
    pltpu.async_copy(h_hbm.at[sidx0], rows0, sem0)
    iload(1, sidx1, didx1, isem1)

    def step(k, carry):
        j0 = 2 * k
        iwait(j0 + 1, sidx1, didx1, isem1)
        pltpu.async_copy(h_hbm.at[sidx1], rows1, sem1)
        gwait(sidx0, rows0, sem0)
        scat(rows0, didx0)
        iload(j0 + 2, sidx0, didx0, isem0)
        gwait(sidx1, rows1, sem1)
        scat(rows1, didx1)
        iwait(j0 + 2, sidx0, didx0, isem0)
        pltpu.async_copy(h_hbm.at[sidx0], rows0, sem0)
        iload(j0 + 3, sidx1, didx1, isem1)
        return carry

    lax.fori_loop(0, 61, step, 0)
    iwait(123, sidx1, didx1, isem1)
    pltpu.async_copy(h_hbm.at[sidx1], rows1, sem1)
    gwait(sidx0, rows0, sem0)
    scat(rows0, didx0)
    iload(124, sidx0, didx0, isem0)
    iwait(124, sidx0, didx0, isem0)
    pltpu.async_copy(h_hbm.at[sidx0], rows0, sem0)
    gwait(sidx1, rows1, sem1)
    scat(rows1, didx1)
    gwait(sidx0, rows0, sem0)
    scat(rows0, didx0)

    plsc.subcore_barrier()
    pltpu.sync_copy(acc.at[pl.ds(sid * ZR, ZR)],
                    out_hbm.at[cid, pl.ds(sid * ZR, ZR)])


@functools.partial(
    pl.kernel,
    out_type=jax.ShapeDtypeStruct((NC, NP, D), jnp.float32),
    mesh=_sc_mesh,
    scratch_types=[
        pltpu.VMEM((NCHUNK, C), jnp.int32),
        pltpu.VMEM((C, D), jnp.float32),
        pltpu.VMEM_SHARED((NP, D), jnp.float32),
    ],
)
def _sc_count(dst_hbm, zero_hbm, out_hbm, didx, ones_v, acc):
    cid = lax.axis_index("c")
    sid = lax.axis_index("s")
    wid = cid * NS + sid
    pltpu.sync_copy(dst_hbm.at[wid], didx)
    pltpu.sync_copy(zero_hbm.at[pl.ds(sid * ZR, ZR)], acc.at[pl.ds(sid * ZR, ZR)])

    one = jnp.ones((16,), jnp.float32)

    def fill(i, carry):
        for c16 in range(D // 16):
            ones_v[i, pl.ds(c16 * 16, 16)] = one
        return carry

    lax.fori_loop(0, C, fill, 0)
    plsc.subcore_barrier()

    def step(j, carry):
        pltpu.sync_copy(ones_v, acc.at[didx.at[j]], add=True)
        return carry

    lax.fori_loop(0, NCHUNK, step, 0)
    plsc.subcore_barrier()
    pltpu.sync_copy(acc.at[pl.ds(sid * ZR, ZR)],
                    out_hbm.at[cid, pl.ds(sid * ZR, ZR)])


_RB = 1000


def _combine_body(has_bn, h, accp, cntp, w1, w2, b, g, be, out):
    s = accp[0] + accp[1]
    c = cntp[0, :, 0:1] + cntp[1, :, 0:1]
    hn = s * (1.0 / jnp.maximum(c, 1.0))
    h2 = (lax.dot_general(h[...], w1[...], (((1,), (1,)), ((), ())),
                          preferred_element_type=jnp.float32)
          + lax.dot_general(hn, w2[...], (((1,), (1,)), ((), ())),
                            preferred_element_type=jnp.float32)
          + b[...])
    nrm = jnp.maximum(jnp.sqrt(jnp.sum(h2 * h2, axis=1, keepdims=True)), 1e-12)
    y = h2 / nrm
    if has_bn:
        y = y * (g[...] / jnp.sqrt(1.0 + 1e-5)) + be[...]
        y = jnp.maximum(y, 0.0)
    out[...] = y


def _combine(h, acc, cnt, w1, w2, b, g, be, has_bn):
    mat = pl.BlockSpec((D, D), lambda i: (0, 0))
    vec = pl.BlockSpec((1, D), lambda i: (0, 0))
    return pl.pallas_call(
        functools.partial(_combine_body, has_bn),
        grid=(N // _RB,),
        in_specs=[
            pl.BlockSpec((_RB, D), lambda i: (i, 0)),
            pl.BlockSpec((NC, _RB, D), lambda i: (0, i, 0)),
            pl.BlockSpec((NC, _RB, 8), lambda i: (0, i, 0)),
            mat, mat, vec, vec, vec,
        ],
        out_specs=pl.BlockSpec((_RB, D), lambda i: (i, 0)),
        out_shape=jax.ShapeDtypeStruct((N, D), jnp.float32),
    )(h, acc, cnt, w1, w2, b[None, :], g[None, :], be[None, :])


def kernel(x, edge_index, W1_0, W2_0, b_0, W1_1, W2_1, b_1, W1_2, W2_2, b_2,
           g_0, be_0, g_1, be_1):
    ei = edge_index.astype(jnp.int32)
    pad = E_PAD - E
    src = ei[0]
    dst = ei[1]
    dst3 = jnp.concatenate([dst, jnp.full((pad,), NP - 1, jnp.int32)]).reshape(
        NW, NCHUNK, C)
    zero_nd = jnp.zeros((NP, D), jnp.float32)

    cnt = _sc_count(dst3, zero_nd)[:, :, :8]

    h = x
    layers = [
        (W1_0, W2_0, b_0, g_0, be_0, True),
        (W1_1, W2_1, b_1, g_1, be_1, True),
        (W1_2, W2_2, b_2, g_1, be_1, False),
    ]
    for w1, w2, b, g, be, has_bn in layers:
        acc = _sc_agg(h, src, dst, zero_nd)
        h = _combine(h, acc, cnt, w1, w2, b, g, be, has_bn)
    return h


# speedup vs baseline: 3.7030x; 1.3511x over previous
import functools

import jax
import jax.numpy as jnp
from jax import lax
from jax.experimental import pallas as pl
from jax.experimental.pallas import tpu as pltpu
from jax.experimental.pallas import tpu_sc as plsc

N = 10000
E = 320000
D = 128

NC = 2
NS = 16
NW = NC * NS
C = 128
NCHUNK = 80
EPW = NCHUNK * C
E_PAD = NW * EPW
NPAIR = NCHUNK // 2
NP = 10240
ZR = NP // NS

_sc_mesh = plsc.VectorSubcoreMesh(
    core_axis_name="c", subcore_axis_name="s", num_cores=NC, num_subcores=NS)


AC = 80
ANCHUNK = 10000 // AC
AEPW = 10000


NBUF = 4


@functools.partial(
    pl.kernel,
    out_type=jax.ShapeDtypeStruct((NC, NP, D), jnp.float32),
    mesh=_sc_mesh,
    scratch_types=(
        [pltpu.VMEM((AC,), jnp.int32)] * NBUF
        + [pltpu.VMEM((AC,), jnp.int32)] * NBUF
        + [pltpu.VMEM((AC, D), jnp.float32)] * NBUF
        + [pltpu.VMEM_SHARED((NP, D), jnp.float32)]
        + [pltpu.SemaphoreType.DMA] * (2 * NBUF)
    ),
)
def _sc_agg(h_hbm, src_hbm, dst_hbm, zero_hbm, out_hbm, *refs):
    sidx = refs[0:NBUF]
    didx = refs[NBUF:2 * NBUF]
    rows = refs[2 * NBUF:3 * NBUF]
    acc = refs[3 * NBUF]
    gsem = refs[3 * NBUF + 1:3 * NBUF + 1 + NBUF]
    isem = refs[3 * NBUF + 1 + NBUF:]
    cid = lax.axis_index("c")
    sid = lax.axis_index("s")
    wid = cid * NS + sid
    pltpu.sync_copy(zero_hbm.at[pl.ds(sid * ZR, ZR)], acc.at[pl.ds(sid * ZR, ZR)])
    plsc.subcore_barrier()

    def iload(j, m):
        e0 = wid * AEPW + j * AC
        pltpu.async_copy(src_hbm.at[pl.ds(e0, AC)], sidx[m], isem[m])
        pltpu.async_copy(dst_hbm.at[pl.ds(e0, AC)], didx[m], isem[m])

    def iwait(j, m):
        e0 = wid * AEPW + j * AC
        pltpu.make_async_copy(src_hbm.at[pl.ds(e0, AC)], sidx[m], isem[m]).wait()
        pltpu.make_async_copy(dst_hbm.at[pl.ds(e0, AC)], didx[m], isem[m]).wait()

    def gissue(m):
        pltpu.async_copy(h_hbm.at[sidx[m]], rows[m], gsem[m])

    def gwait(m):
        pltpu.make_async_copy(h_hbm.at[sidx[m]], rows[m], gsem[m]).wait()

    def scat(m):
        pltpu.sync_copy(rows[m], acc.at[didx[m]], add=True)

    for j in range(NBUF):
        iload(j, j)
    for j in range(NBUF - 1):
        iwait(j, j)
        gissue(j)

    def quad(k, carry):
        j = 4 * k
        for m in range(4):
            gwait(m)
            scat(m)
            iload(j + m + NBUF, m)
            iwait(j + m + NBUF - 1, (m + NBUF - 1) % NBUF)
            gissue((m + NBUF - 1) % NBUF)
        return carry

    lax.fori_loop(0, (ANCHUNK - 5) // 4, quad, 0)
    gwait(0); scat(0)
    iload(124, 0)
    iwait(123, 3)
    gissue(3)
    gwait(1); scat(1)
    iwait(124, 0)
    gissue(0)
    gwait(2); scat(2)
    gwait(3); scat(3)
    gwait(0); scat(0)

    plsc.subcore_barrier()
    pltpu.sync_copy(acc.at[pl.ds(sid * ZR, ZR)],
                    out_hbm.at[cid, pl.ds(sid * ZR, ZR)])


@functools.partial(
    pl.kernel,
    out_type=jax.ShapeDtypeStruct((NC, NP, D), jnp.float32),
    mesh=_sc_mesh,
    scratch_types=[
        pltpu.VMEM((NCHUNK, C), jnp.int32),
        pltpu.VMEM((C, D), jnp.float32),
        pltpu.VMEM_SHARED((NP, D), jnp.float32),
    ],
)
def _sc_count(dst_hbm, zero_hbm, out_hbm, didx, ones_v, acc):
    cid = lax.axis_index("c")
    sid = lax.axis_index("s")
    wid = cid * NS + sid
    pltpu.sync_copy(dst_hbm.at[wid], didx)
    pltpu.sync_copy(zero_hbm.at[pl.ds(sid * ZR, ZR)], acc.at[pl.ds(sid * ZR, ZR)])

    one = jnp.ones((16,), jnp.float32)

    def fill(i, carry):
        for c16 in range(D // 16):
            ones_v[i, pl.ds(c16 * 16, 16)] = one
        return carry

    lax.fori_loop(0, C, fill, 0)
    plsc.subcore_barrier()

    def step(j, carry):
        pltpu.sync_copy(ones_v, acc.at[didx.at[j]], add=True)
        return carry

    lax.fori_loop(0, NCHUNK, step, 0)
    plsc.subcore_barrier()
    pltpu.sync_copy(acc.at[pl.ds(sid * ZR, ZR)],
                    out_hbm.at[cid, pl.ds(sid * ZR, ZR)])


_RB = 1000


def _combine_body(has_bn, h, accp, cntp, w1, w2, b, g, be, out):
    s = accp[0] + accp[1]
    c = cntp[0, :, 0:1] + cntp[1, :, 0:1]
    hn = s * (1.0 / jnp.maximum(c, 1.0))
    h2 = (lax.dot_general(h[...], w1[...], (((1,), (1,)), ((), ())),
                          preferred_element_type=jnp.float32)
          + lax.dot_general(hn, w2[...], (((1,), (1,)), ((), ())),
                            preferred_element_type=jnp.float32)
          + b[...])
    nrm = jnp.maximum(jnp.sqrt(jnp.sum(h2 * h2, axis=1, keepdims=True)), 1e-12)
    y = h2 / nrm
    if has_bn:
        y = y * (g[...] / jnp.sqrt(1.0 + 1e-5)) + be[...]
        y = jnp.maximum(y, 0.0)
    out[...] = y


def _combine(h, acc, cnt, w1, w2, b, g, be, has_bn):
    mat = pl.BlockSpec((D, D), lambda i: (0, 0))
    vec = pl.BlockSpec((1, D), lambda i: (0, 0))
    return pl.pallas_call(
        functools.partial(_combine_body, has_bn),
        grid=(N // _RB,),
        in_specs=[
            pl.BlockSpec((_RB, D), lambda i: (i, 0)),
            pl.BlockSpec((NC, _RB, D), lambda i: (0, i, 0)),
            pl.BlockSpec((NC, _RB, 8), lambda i: (0, i, 0)),
            mat, mat, vec, vec, vec,
        ],
        out_specs=pl.BlockSpec((_RB, D), lambda i: (i, 0)),
        out_shape=jax.ShapeDtypeStruct((N, D), jnp.float32),
    )(h, acc, cnt, w1, w2, b[None, :], g[None, :], be[None, :])


def kernel(x, edge_index, W1_0, W2_0, b_0, W1_1, W2_1, b_1, W1_2, W2_2, b_2,
           g_0, be_0, g_1, be_1):
    ei = edge_index.astype(jnp.int32)
    pad = E_PAD - E
    src = ei[0]
    dst = ei[1]
    dst3 = jnp.concatenate([dst, jnp.full((pad,), NP - 1, jnp.int32)]).reshape(
        NW, NCHUNK, C)
    zero_nd = jnp.zeros((NP, D), jnp.float32)

    cnt = _sc_count(dst3, zero_nd)[:, :, :8]

    h = x
    layers = [
        (W1_0, W2_0, b_0, g_0, be_0, True),
        (W1_1, W2_1, b_1, g_1, be_1, True),
        (W1_2, W2_2, b_2, g_1, be_1, False),
    ]
    for w1, w2, b, g, be, has_bn in layers:
        acc = _sc_agg(h, src, dst, zero_nd)
        h = _combine(h, acc, cnt, w1, w2, b, g, be, has_bn)
    return h


# speedup vs baseline: 3.7782x; 1.0203x over previous
import functools

import jax
import jax.numpy as jnp
from jax import lax
from jax.experimental import pallas as pl
from jax.experimental.pallas import tpu as pltpu
from jax.experimental.pallas import tpu_sc as plsc

N = 10000
E = 320000
D = 128

NC = 2
NS = 16
NW = NC * NS
C = 128
NCHUNK = 80
EPW = NCHUNK * C
E_PAD = NW * EPW
NPAIR = NCHUNK // 2
NP = 10240
ZR = NP // NS

_sc_mesh = plsc.VectorSubcoreMesh(
    core_axis_name="c", subcore_axis_name="s", num_cores=NC, num_subcores=NS)


AC = 80
ANCHUNK = 10000 // AC
AEPW = 10000


NBUF = 4


@functools.partial(
    pl.kernel,
    out_type=jax.ShapeDtypeStruct((NC, NP, D), jnp.float32),
    mesh=_sc_mesh,
    scratch_types=(
        [pltpu.VMEM((AC,), jnp.int32)] * NBUF
        + [pltpu.VMEM((AC,), jnp.int32)] * NBUF
        + [pltpu.VMEM((AC, D), jnp.float32)] * NBUF
        + [pltpu.VMEM_SHARED((NP, D), jnp.float32)]
        + [pltpu.SemaphoreType.DMA] * (2 * NBUF + 1)
    ),
)
def _sc_agg(h_hbm, src_hbm, dst_hbm, zero_hbm, out_hbm, *refs):
    sidx = refs[0:NBUF]
    didx = refs[NBUF:2 * NBUF]
    rows = refs[2 * NBUF:3 * NBUF]
    acc = refs[3 * NBUF]
    gsem = refs[3 * NBUF + 1:3 * NBUF + 1 + NBUF]
    isem = refs[3 * NBUF + 1 + NBUF:3 * NBUF + 1 + 2 * NBUF]
    zsem = refs[3 * NBUF + 1 + 2 * NBUF]
    cid = lax.axis_index("c")
    sid = lax.axis_index("s")
    wid = cid * NS + sid
    zcp = pltpu.async_copy(zero_hbm.at[pl.ds(sid * ZR, ZR)],
                           acc.at[pl.ds(sid * ZR, ZR)], zsem)

    def iload(j, m):
        e0 = wid * AEPW + j * AC
        pltpu.async_copy(src_hbm.at[pl.ds(e0, AC)], sidx[m], isem[m])
        pltpu.async_copy(dst_hbm.at[pl.ds(e0, AC)], didx[m], isem[m])

    def iwait(j, m):
        e0 = wid * AEPW + j * AC
        pltpu.make_async_copy(src_hbm.at[pl.ds(e0, AC)], sidx[m], isem[m]).wait()
        pltpu.make_async_copy(dst_hbm.at[pl.ds(e0, AC)], didx[m], isem[m]).wait()

    def gissue(m):
        pltpu.async_copy(h_hbm.at[sidx[m]], rows[m], gsem[m])

    def gwait(m):
        pltpu.make_async_copy(h_hbm.at[sidx[m]], rows[m], gsem[m]).wait()

    def scat(m):
        pltpu.sync_copy(rows[m], acc.at[didx[m]], add=True)

    for j in range(NBUF):
        iload(j, j)
    for j in range(NBUF - 1):
        iwait(j, j)
        gissue(j)
    zcp.wait()
    plsc.subcore_barrier()

    def quad(k, carry):
        j = 4 * k
        for m in range(4):
            gwait(m)
            scat(m)
            iload(j + m + NBUF, m)
            iwait(j + m + NBUF - 1, (m + NBUF - 1) % NBUF)
            gissue((m + NBUF - 1) % NBUF)
        return carry

    lax.fori_loop(0, (ANCHUNK - 5) // 4, quad, 0)
    gwait(0); scat(0)
    iload(124, 0)
    iwait(123, 3)
    gissue(3)
    gwait(1); scat(1)
    iwait(124, 0)
    gissue(0)
    gwait(2); scat(2)
    gwait(3); scat(3)
    gwait(0); scat(0)

    plsc.subcore_barrier()
    pltpu.sync_copy(acc.at[pl.ds(sid * ZR, ZR)],
                    out_hbm.at[cid, pl.ds(sid * ZR, ZR)])


@functools.partial(
    pl.kernel,
    out_type=jax.ShapeDtypeStruct((NC, NP, D), jnp.float32),
    mesh=_sc_mesh,
    scratch_types=[
        pltpu.VMEM((NCHUNK, C), jnp.int32),
        pltpu.VMEM((C, D), jnp.float32),
        pltpu.VMEM_SHARED((NP, D), jnp.float32),
        pltpu.SemaphoreType.DMA,
    ],
)
def _sc_count(dst_hbm, zero_hbm, out_hbm, didx, ones_v, acc, zsem):
    cid = lax.axis_index("c")
    sid = lax.axis_index("s")
    wid = cid * NS + sid
    zcp = pltpu.async_copy(zero_hbm.at[pl.ds(sid * ZR, ZR)],
                           acc.at[pl.ds(sid * ZR, ZR)], zsem)
    pltpu.sync_copy(dst_hbm.at[wid], didx)

    one = jnp.ones((16,), jnp.float32)

    def fill(i, carry):
        for c16 in range(D // 16):
            ones_v[i, pl.ds(c16 * 16, 16)] = one
        return carry

    lax.fori_loop(0, C, fill, 0)
    zcp.wait()
    plsc.subcore_barrier()

    def step(j, carry):
        pltpu.sync_copy(ones_v, acc.at[didx.at[j]], add=True)
        return carry

    lax.fori_loop(0, NCHUNK, step, 0)
    plsc.subcore_barrier()
    pltpu.sync_copy(acc.at[pl.ds(sid * ZR, ZR)],
                    out_hbm.at[cid, pl.ds(sid * ZR, ZR)])


_RB = 1000


def _combine_body(has_bn, h, accp, cntp, w1, w2, b, g, be, out):
    s = accp[0] + accp[1]
    c = cntp[0, :, 0:1] + cntp[1, :, 0:1]
    hn = s * (1.0 / jnp.maximum(c, 1.0))
    h2 = (lax.dot_general(h[...], w1[...], (((1,), (1,)), ((), ())),
                          preferred_element_type=jnp.float32)
          + lax.dot_general(hn, w2[...], (((1,), (1,)), ((), ())),
                            preferred_element_type=jnp.float32)
          + b[...])
    nrm = jnp.maximum(jnp.sqrt(jnp.sum(h2 * h2, axis=1, keepdims=True)), 1e-12)
    y = h2 / nrm
    if has_bn:
        y = y * (g[...] / jnp.sqrt(1.0 + 1e-5)) + be[...]
        y = jnp.maximum(y, 0.0)
    out[...] = y


def _combine(h, acc, cnt, w1, w2, b, g, be, has_bn):
    mat = pl.BlockSpec((D, D), lambda i: (0, 0))
    vec = pl.BlockSpec((1, D), lambda i: (0, 0))
    return pl.pallas_call(
        functools.partial(_combine_body, has_bn),
        grid=(N // _RB,),
        in_specs=[
            pl.BlockSpec((_RB, D), lambda i: (i, 0)),
            pl.BlockSpec((NC, _RB, D), lambda i: (0, i, 0)),
            pl.BlockSpec((NC, _RB, 8), lambda i: (0, i, 0)),
            mat, mat, vec, vec, vec,
        ],
        out_specs=pl.BlockSpec((_RB, D), lambda i: (i, 0)),
        out_shape=jax.ShapeDtypeStruct((N, D), jnp.float32),
    )(h, acc, cnt, w1, w2, b[None, :], g[None, :], be[None, :])


def kernel(x, edge_index, W1_0, W2_0, b_0, W1_1, W2_1, b_1, W1_2, W2_2, b_2,
           g_0, be_0, g_1, be_1):
    ei = edge_index.astype(jnp.int32)
    pad = E_PAD - E
    src = ei[0]
    dst = ei[1]
    dst3 = jnp.concatenate([dst, jnp.full((pad,), NP - 1, jnp.int32)]).reshape(
        NW, NCHUNK, C)
    zero_nd = jnp.zeros((NP, D), jnp.float32)

    cnt = _sc_count(dst3, zero_nd)[:, :, :8]

    h = x
    layers = [
        (W1_0, W2_0, b_0, g_0, be_0, True),
        (W1_1, W2_1, b_1, g_1, be_1, True),
        (W1_2, W2_2, b_2, g_1, be_1, False),
    ]
    for w1, w2, b, g, be, has_bn in layers:
        acc = _sc_agg(h, src, dst, zero_nd)
        h = _combine(h, acc, cnt, w1, w2, b, g, be, has_bn)
    return h
